# baseline (speedup 1.0000x reference)
import functools

import jax
import jax.numpy as jnp
import numpy as np
from jax import lax
from jax.experimental import pallas as pl
from jax.experimental.pallas import tpu as pltpu
from jax.experimental.pallas import tpu_sc as plsc

_NC = 2
_NS = 16
_NW = _NC * _NS
_V = 1000000
_D = 64
_B = 4096
_L = 200
_VB_FULL = _V // 128
_VB_MAIN = (_VB_FULL // _NW) * _NW
_R_ROWS = _VB_FULL * 64 + 64


def _positional_encoding_np(max_len, dim):
    position = np.arange(max_len, dtype=np.float64)[:, None]
    i = np.arange(0, dim, 2, dtype=np.float64)[None, :] / dim
    exp_term = 10000.0 ** i
    enc = np.zeros((max_len, dim), dtype=np.float32)
    enc[:, 0::2] = np.sin(position / exp_term)
    enc[:, 1::2] = np.cos(position / exp_term)
    return enc


_CP = pltpu.CompilerParams(use_tc_tiling_on_sc=True, needs_layout_passes=False)
_MESH = dict(core_axis_name="c", subcore_axis_name="s", num_cores=_NC)


@jax.jit
def _pack_table(tt, tailp):
    mesh = plsc.VectorSubcoreMesh(**_MESH)

    @functools.partial(
        pl.kernel,
        out_type=jax.ShapeDtypeStruct((_R_ROWS, 128), jnp.float32),
        mesh=mesh,
        scratch_types=[
            pltpu.VMEM((64, 128), jnp.float32),
            pltpu.VMEM((64, 128), jnp.float32),
            pltpu.VMEM((64, 128), jnp.float32),
            pltpu.VMEM((64, 128), jnp.float32),
            pltpu.SemaphoreType.DMA,
            pltpu.SemaphoreType.DMA,
            pltpu.SemaphoreType.DMA,
            pltpu.SemaphoreType.DMA,
        ],
        compiler_params=_CP,
    )
    def k1(tt_hbm, tailp_hbm, r_hbm, s0, s1, d0, d1, rs0, rs1, ws0, ws1):
        src = (s0, s1)
        dst = (d0, d1)
        rsem = (rs0, rs1)
        wsem = (ws0, ws1)
        wid = lax.axis_index("s") * _NC + lax.axis_index("c")
        iot = lax.broadcasted_iota(jnp.int32, (16,), 0)
        par64 = lax.bitwise_and(iot, 1) * 64
        colj = [iot + g * 16 for g in range(8)]
        pvec = [lax.shift_right_logical(colj[g], 1) for g in range(8)]

        def transpose(sbuf, dbuf):
            @plsc.parallel_loop(0, 16)
            def _(k):
                dm = lax.bitwise_and(iot + k, 15)
                for d0 in range(4):
                    dmd = dm + d0 * 16
                    cs = par64 + dmd
                    for g in range(8):
                        v = plsc.load_gather(sbuf, [dmd, colj[g]])
                        plsc.store_scatter(dbuf, [pvec[g], cs], v)

        def fire_read(k, bu):
            vb = wid + k * _NW
            pltpu.async_copy(
                tt_hbm.at[:, pl.ds(vb * 128, 128)], src[bu], rsem[bu]
            )

        def wait_read(bu):
            pltpu.make_async_copy(
                tt_hbm.at[:, pl.ds(0, 128)], src[bu], rsem[bu]
            ).wait()

        def fire_write(k, bu):
            vb = wid + k * _NW
            pltpu.async_copy(
                dst[bu], r_hbm.at[pl.ds(vb * 64, 64)], wsem[bu]
            )

        def wait_write(bu):
            pltpu.make_async_copy(
                dst[bu], r_hbm.at[pl.ds(0, 64)], wsem[bu]
            ).wait()

        n = _VB_MAIN // _NW
        fire_read(0, 0)

        @pl.loop(0, n, step=2)
        def _(k0):
            for bu in range(2):
                k = k0 + bu
                nb = 1 - bu
                wait_read(bu)
                @pl.when(k + 1 < n)
                def _():
                    fire_read(k + 1, nb)
                @pl.when(k >= 2)
                def _():
                    wait_write(bu)
                transpose(src[bu], dst[bu])
                fire_write(k, bu)

        wait_write(0)
        wait_write(1)

        @pl.when(wid < _VB_FULL - _VB_MAIN)
        def _():
            vb = _VB_MAIN + wid
            pltpu.sync_copy(tt_hbm.at[:, pl.ds(vb * 128, 128)], s0)
            transpose(s0, d0)
            pltpu.sync_copy(d0, r_hbm.at[pl.ds(vb * 64, 64)])

        @pl.when(wid == _NW - 1)
        def _():
            pltpu.sync_copy(tailp_hbm, s1)
            transpose(s1, d1)
            pltpu.sync_copy(d1, r_hbm.at[pl.ds(_VB_FULL * 64, 64)])

    return k1(tt, tailp)


@jax.jit
def _gather_pe(xflat_t, r, pet):
    mesh = plsc.VectorSubcoreMesh(**_MESH)
    n_units = (_L // 8) * (_B // 128) // _NW
    cp_lin = pltpu.CompilerParams(
        use_tc_tiling_on_sc=False, needs_layout_passes=False
    )

    @functools.partial(
        pl.kernel,
        out_type=jax.ShapeDtypeStruct((_L, _D, _B), jnp.float32),
        mesh=mesh,
        scratch_types=[
            pltpu.VMEM((1024,), jnp.int32),
            pltpu.VMEM((1024,), jnp.int32),
            pltpu.VMEM((8192,), jnp.float32),
            pltpu.VMEM((128, 64), jnp.float32),
            pltpu.VMEM((128, 64), jnp.float32),
            pltpu.VMEM((64, 128), jnp.float32),
            pltpu.VMEM((64, 128), jnp.float32),
            pltpu.SemaphoreType.DMA,
            pltpu.SemaphoreType.DMA,
            pltpu.SemaphoreType.DMA,
            pltpu.SemaphoreType.DMA,
            pltpu.SemaphoreType.DMA,
            pltpu.SemaphoreType.DMA,
        ],
        compiler_params=cp_lin,
    )
    def k2(x_hbm, r_hbm, pet_hbm, out_hbm, ti_v, pidx_v, pet_v,
           rows0, rows1, ob0, ob1, isem, psem, gs0, gs1, ws0, ws1):
        rows = (rows0, rows1)
        gsem = (gs0, gs1)
        obuf = (ob0, ob1)
        wsem = (ws0, ws1)
        wid = lax.axis_index("s") * _NC + lax.axis_index("c")
        iot = lax.broadcasted_iota(jnp.int32, (16,), 0)
        rowi = [iot + g * 16 for g in range(8)]

        def fire_gather(li):
            pltpu.async_copy(
                r_hbm.at[ti_v.at[pl.ds(li * 128, 128)]],
                rows[li % 2],
                gsem[li % 2],
            )

        def wait_gather(li):
            pltpu.make_async_copy(
                r_hbm.at[pl.ds(0, 128)], rows[li % 2], gsem[li % 2]
            ).wait()

        def wait_store(bu, lg_bb_dummy=None):
            pltpu.make_async_copy(
                obuf[bu], out_hbm.at[0, :, pl.ds(0, 128)], wsem[bu]
            ).wait()

        @pl.loop(0, n_units)
        def _(t):
            u = wid * n_units + t
            lg = lax.div(u, jnp.int32(_B // 128))
            bb = lax.rem(u, jnp.int32(_B // 128))
            for li in range(8):
                pltpu.async_copy(
                    x_hbm.at[pl.ds((lg * 8 + li) * _B + bb * 128, 128)],
                    ti_v.at[pl.ds(li * 128, 128)],
                    isem,
                )
            pltpu.async_copy(
                pet_hbm.at[pl.ds(lg * 8192, 8192)], pet_v, psem
            )
            for li in range(8):
                pltpu.make_async_copy(
                    x_hbm.at[pl.ds(0, 128)],
                    ti_v.at[pl.ds(li * 128, 128)],
                    isem,
                ).wait()
            pltpu.make_async_copy(
                pet_hbm.at[pl.ds(0, 8192)], pet_v, psem
            ).wait()
            fire_gather(0)

            for li in range(8):
                bu = li % 2
                l = lg * 8 + li
                wait_gather(li)
                if li + 1 < 8:
                    fire_gather(li + 1)
                if li >= 2:
                    wait_store(bu)
                else:
                    @pl.when(t > 0)
                    def _():
                        wait_store(bu)
                @plsc.parallel_loop(0, 16)
                def _(k):
                    dm = lax.bitwise_and(iot + k, 15)
                    for d0 in range(4):
                        dmd = dm + d0 * 16
                        pev = pet_v[pl.ds(li * 1024 + k * 64 + d0 * 16, 16)]
                        for g in range(8):
                            v = plsc.load_gather(rows[bu], [rowi[g], dmd])
                            plsc.store_scatter(
                                obuf[bu], [dmd, rowi[g]], v + pev
                            )

                pltpu.async_copy(
                    obuf[bu],
                    out_hbm.at[l, :, pl.ds(bb * 128, 128)],
                    wsem[bu],
                )

        wait_store(0)
        wait_store(1)

    return k2(xflat_t, r, pet)


def kernel(x, table):
    b, l = x.shape
    v, d = table.shape
    pe = _positional_encoding_np(l, d)
    lane_k = (np.arange(16)[:, None] + np.arange(16)[None, :]) % 16
    didx = (np.arange(4)[:, None] * 16)[None, :, :] + lane_k[:, None, :]
    pet = jnp.asarray(pe[:, didx].reshape(-1))
    tt = jnp.swapaxes(table, 0, 1)
    tailp = jnp.pad(jnp.swapaxes(table[_VB_FULL * 128:, :], 0, 1),
                    ((0, 0), (0, 128 - (v - _VB_FULL * 128))))
    xflat_t = jnp.swapaxes(x, 0, 1).reshape(-1)
    r = _pack_table(tt, tailp)
    out5 = _gather_pe(xflat_t, jnp.reshape(r, (_R_ROWS * 2, _D)), pet)
    return jnp.transpose(out5, (2, 0, 1))


# speedup vs baseline: 1.2769x; 1.2769x over previous
import functools

import jax
import jax.numpy as jnp
import numpy as np
from jax import lax
from jax.experimental import pallas as pl
from jax.experimental.pallas import tpu as pltpu
from jax.experimental.pallas import tpu_sc as plsc

_NC = 2
_NS = 16
_NW = _NC * _NS
_V = 1000000
_D = 64
_B = 4096
_L = 200
_VB_FULL = _V // 128
_VB_MAIN = (_VB_FULL // _NW) * _NW
_R_ROWS = _VB_FULL * 64 + 64


def _positional_encoding_np(max_len, dim):
    position = np.arange(max_len, dtype=np.float64)[:, None]
    i = np.arange(0, dim, 2, dtype=np.float64)[None, :] / dim
    exp_term = 10000.0 ** i
    enc = np.zeros((max_len, dim), dtype=np.float32)
    enc[:, 0::2] = np.sin(position / exp_term)
    enc[:, 1::2] = np.cos(position / exp_term)
    return enc


_CP = pltpu.CompilerParams(use_tc_tiling_on_sc=True, needs_layout_passes=False)
_MESH = dict(core_axis_name="c", subcore_axis_name="s", num_cores=_NC)


@jax.jit
def _pack_table(tt, tailp):
    mesh = plsc.VectorSubcoreMesh(**_MESH)

    @functools.partial(
        pl.kernel,
        out_type=jax.ShapeDtypeStruct((_R_ROWS, 128), jnp.float32),
        mesh=mesh,
        scratch_types=[
            pltpu.VMEM((64, 128), jnp.float32),
            pltpu.VMEM((64, 128), jnp.float32),
            pltpu.VMEM((64, 128), jnp.float32),
            pltpu.VMEM((64, 128), jnp.float32),
            pltpu.SemaphoreType.DMA,
            pltpu.SemaphoreType.DMA,
            pltpu.SemaphoreType.DMA,
            pltpu.SemaphoreType.DMA,
        ],
        compiler_params=_CP,
    )
    def k1(tt_hbm, tailp_hbm, r_hbm, s0, s1, d0, d1, rs0, rs1, ws0, ws1):
        src = (s0, s1)
        dst = (d0, d1)
        rsem = (rs0, rs1)
        wsem = (ws0, ws1)
        wid = lax.axis_index("s") * _NC + lax.axis_index("c")
        iot = lax.broadcasted_iota(jnp.int32, (16,), 0)
        par64 = lax.bitwise_and(iot, 1) * 64
        colj = [iot + g * 16 for g in range(8)]
        pvec = [lax.shift_right_logical(colj[g], 1) for g in range(8)]

        def transpose(sbuf, dbuf):
            @plsc.parallel_loop(0, 16)
            def _(k):
                dm = lax.bitwise_and(iot + k, 15)
                for d0 in range(4):
                    dmd = dm + d0 * 16
                    cs = par64 + dmd
                    for g in range(8):
                        v = plsc.load_gather(sbuf, [dmd, colj[g]])
                        plsc.store_scatter(dbuf, [pvec[g], cs], v)

        def fire_read(k, bu):
            vb = wid + k * _NW
            pltpu.async_copy(
                tt_hbm.at[:, pl.ds(vb * 128, 128)], src[bu], rsem[bu]
            )

        def wait_read(bu):
            pltpu.make_async_copy(
                tt_hbm.at[:, pl.ds(0, 128)], src[bu], rsem[bu]
            ).wait()

        def fire_write(k, bu):
            vb = wid + k * _NW
            pltpu.async_copy(
                dst[bu], r_hbm.at[pl.ds(vb * 64, 64)], wsem[bu]
            )

        def wait_write(bu):
            pltpu.make_async_copy(
                dst[bu], r_hbm.at[pl.ds(0, 64)], wsem[bu]
            ).wait()

        n = _VB_MAIN // _NW
        fire_read(0, 0)

        @pl.loop(0, n, step=2)
        def _(k0):
            for bu in range(2):
                k = k0 + bu
                nb = 1 - bu
                wait_read(bu)
                @pl.when(k + 1 < n)
                def _():
                    fire_read(k + 1, nb)
                @pl.when(k >= 2)
                def _():
                    wait_write(bu)
                transpose(src[bu], dst[bu])
                fire_write(k, bu)

        wait_write(0)
        wait_write(1)

        @pl.when(wid < _VB_FULL - _VB_MAIN)
        def _():
            vb = _VB_MAIN + wid
            pltpu.sync_copy(tt_hbm.at[:, pl.ds(vb * 128, 128)], s0)
            transpose(s0, d0)
            pltpu.sync_copy(d0, r_hbm.at[pl.ds(vb * 64, 64)])

        @pl.when(wid == _NW - 1)
        def _():
            pltpu.sync_copy(tailp_hbm, s1)
            transpose(s1, d1)
            pltpu.sync_copy(d1, r_hbm.at[pl.ds(_VB_FULL * 64, 64)])

    return k1(tt, tailp)


@jax.jit
def _gather_pe(xflat_t, r, pet):
    mesh = plsc.VectorSubcoreMesh(**_MESH)
    n_units = (_L // 8) * (_B // 128) // _NW

    @functools.partial(
        pl.kernel,
        out_type=jax.ShapeDtypeStruct((_L, _D, _B), jnp.float32),
        mesh=mesh,
        scratch_types=[
            pltpu.VMEM((1024,), jnp.int32),
            pltpu.VMEM((1024,), jnp.int32),
            pltpu.VMEM((8192,), jnp.float32),
            pltpu.VMEM((128, 128), jnp.float32),
            pltpu.VMEM((128, 128), jnp.float32),
            pltpu.VMEM((128, 128), jnp.float32),
            pltpu.VMEM((128, 128), jnp.float32),
            pltpu.VMEM((64, 128), jnp.float32),
            pltpu.VMEM((64, 128), jnp.float32),
            pltpu.SemaphoreType.DMA,
            pltpu.SemaphoreType.DMA,
            pltpu.SemaphoreType.DMA,
            pltpu.SemaphoreType.DMA,
            pltpu.SemaphoreType.DMA,
            pltpu.SemaphoreType.DMA,
            pltpu.SemaphoreType.DMA,
            pltpu.SemaphoreType.DMA,
        ],
        compiler_params=_CP,
    )
    def k2(x_hbm, r_hbm, pet_hbm, out_hbm, ti_v, pidx_v, pet_v,
           rows0, rows1, rows2, rows3, ob0, ob1, isem, psem,
           gs0, gs1, gs2, gs3, ws0, ws1):
        rows = (rows0, rows1, rows2, rows3)
        gsem = (gs0, gs1, gs2, gs3)
        obuf = (ob0, ob1)
        wsem = (ws0, ws1)
        wid = lax.axis_index("s") * _NC + lax.axis_index("c")
        iot = lax.broadcasted_iota(jnp.int32, (16,), 0)
        rowi = [iot + g * 16 for g in range(8)]

        def fire_gather(li):
            pltpu.async_copy(
                r_hbm.at[pidx_v.at[pl.ds(li * 128, 128)]],
                rows[li % 4],
                gsem[li % 4],
            )

        def wait_gather(li):
            pltpu.make_async_copy(
                r_hbm.at[pl.ds(0, 128)], rows[li % 4], gsem[li % 4]
            ).wait()

        def wait_store(bu, lg_bb_dummy=None):
            pltpu.make_async_copy(
                obuf[bu], out_hbm.at[0, :, pl.ds(0, 128)], wsem[bu]
            ).wait()

        @pl.loop(0, n_units)
        def _(t):
            u = wid * n_units + t
            lg = lax.div(u, jnp.int32(_B // 128))
            bb = lax.rem(u, jnp.int32(_B // 128))
            for li in range(8):
                pltpu.async_copy(
                    x_hbm.at[pl.ds((lg * 8 + li) * _B + bb * 128, 128)],
                    ti_v.at[pl.ds(li * 128, 128)],
                    isem,
                )
            pltpu.async_copy(
                pet_hbm.at[pl.ds(lg * 8192, 8192)], pet_v, psem
            )
            for li in range(8):
                pltpu.make_async_copy(
                    x_hbm.at[pl.ds(0, 128)],
                    ti_v.at[pl.ds(li * 128, 128)],
                    isem,
                ).wait()
            @plsc.parallel_loop(0, 64, unroll=4)
            def _(q):
                pidx_v[pl.ds(q * 16, 16)] = lax.shift_right_logical(
                    ti_v[pl.ds(q * 16, 16)], 1
                )
            pltpu.make_async_copy(
                pet_hbm.at[pl.ds(0, 8192)], pet_v, psem
            ).wait()
            fire_gather(0)
            fire_gather(1)
            fire_gather(2)

            for li in range(8):
                bu = li % 2
                rb = li % 4
                l = lg * 8 + li
                wait_gather(li)
                if li + 3 < 8:
                    fire_gather(li + 3)
                if li >= 2:
                    wait_store(bu)
                else:
                    @pl.when(t > 0)
                    def _():
                        wait_store(bu)
                parS = []
                for g in range(8):
                    tv = ti_v[pl.ds(li * 128 + g * 16, 16)]
                    parS.append(lax.bitwise_and(tv, 1) * 64)

                @plsc.parallel_loop(0, 16)
                def _(k):
                    dm = lax.bitwise_and(iot + k, 15)
                    for d0 in range(4):
                        dmd = dm + d0 * 16
                        pev = pet_v[pl.ds(li * 1024 + k * 64 + d0 * 16, 16)]
                        for g in range(8):
                            v = plsc.load_gather(
                                rows[rb], [rowi[g], parS[g] + dmd]
                            )
                            plsc.store_scatter(
                                obuf[bu], [dmd, rowi[g]], v + pev
                            )

                pltpu.async_copy(
                    obuf[bu],
                    out_hbm.at[l, :, pl.ds(bb * 128, 128)],
                    wsem[bu],
                )

        wait_store(0)
        wait_store(1)

    return k2(xflat_t, r, pet)


def kernel(x, table):
    b, l = x.shape
    v, d = table.shape
    pe = _positional_encoding_np(l, d)
    lane_k = (np.arange(16)[:, None] + np.arange(16)[None, :]) % 16
    didx = (np.arange(4)[:, None] * 16)[None, :, :] + lane_k[:, None, :]
    pet = jnp.asarray(pe[:, didx].reshape(-1))
    tt = jnp.swapaxes(table, 0, 1)
    tailp = jnp.pad(jnp.swapaxes(table[_VB_FULL * 128:, :], 0, 1),
                    ((0, 0), (0, 128 - (v - _VB_FULL * 128))))
    xflat_t = jnp.swapaxes(x, 0, 1).reshape(-1)
    r = _pack_table(tt, tailp)
    out5 = _gather_pe(xflat_t, r, pet)
    return jnp.transpose(out5, (2, 0, 1))
